# baseline (speedup 1.0000x reference)
import functools

import jax
import jax.numpy as jnp
from jax import lax
from jax.experimental import pallas as pl
from jax.experimental.pallas import tpu as pltpu
from jax.experimental.pallas import tpu_sc as plsc

_f32 = jnp.float32

_NTILES = 16
_RCH = 128
_ECH = 128

_CORRECTION_LAYERS = 50
_CORRECTION_ALPHA = 0.5
_SMOOTHING_LAYERS = 50
_SMOOTHING_ALPHA = 0.8
_SCALE = 1.0


def _make_deg_kernel(n_pad, nech):
    rt = n_pad // _NTILES
    mesh = plsc.VectorSubcoreMesh(
        core_axis_name="c", subcore_axis_name="s", num_cores=1, num_subcores=16)

    def body(srcdst_hbm, ones_hbm, zer_hbm, deg_hbm, deg_sh, dst_v, ones_v):
        tid = lax.axis_index("s")
        r0 = tid * rt
        pltpu.sync_copy(srcdst_hbm.at[tid], dst_v)
        pltpu.sync_copy(ones_hbm, ones_v)
        pltpu.sync_copy(zer_hbm, deg_sh.at[pl.ds(r0, rt)])
        plsc.subcore_barrier()

        def eloop(c, carry):
            pltpu.sync_copy(ones_v, deg_sh.at[dst_v.at[c, 1]], add=True)
            return carry

        lax.fori_loop(0, nech, eloop, None)
        plsc.subcore_barrier()
        pltpu.sync_copy(deg_sh.at[pl.ds(r0, rt)], deg_hbm.at[pl.ds(r0, rt)])

    return pl.kernel(
        body,
        out_type=jax.ShapeDtypeStruct((n_pad,), _f32),
        mesh=mesh,
        compiler_params=pltpu.CompilerParams(use_tc_tiling_on_sc=False),
        scratch_types=[
            pltpu.VMEM_SHARED((n_pad,), _f32),
            pltpu.VMEM((nech, 2, _ECH), jnp.int32),
            pltpu.VMEM((_ECH,), _f32),
        ],
    )


_NBUF = 8
_NSC = 2


def _make_prop_kernel(n_pad, c, nech, iters, alpha, lo, hi):
    ch = c // _NSC
    rt = n_pad // _NTILES
    nrch = rt // _RCH
    assert nech % _NBUF == 0
    mesh = plsc.VectorSubcoreMesh(
        core_axis_name="c", subcore_axis_name="s",
        num_cores=_NSC, num_subcores=16)

    def body(srcdst_hbm, u0_hbm, resdis_hbm, zer_hbm, out_hbm, u_hbm,
             s_sh, idx_v, gbuf, zbuf, sbuf, rdbuf,
             gsem, ssem):
        cid = lax.axis_index("c")
        tid = lax.axis_index("s")
        row0 = tid * rt
        off = cid * n_pad

        def _gather_cp(ck, b):
            return pltpu.make_async_copy(
                u_hbm.at[idx_v.at[ck, 0]], gbuf.at[b], gsem.at[b])

        def _scatter_cp(ck, b):
            return pltpu.make_async_copy(
                gbuf.at[b], s_sh.at[idx_v.at[ck, 1]], ssem.at[b])

        pltpu.sync_copy(srcdst_hbm.at[tid], idx_v)
        pltpu.sync_copy(zer_hbm, zbuf)

        def off_loop(ck, c2):
            for i in range(_ECH // 16):
                sl = pl.ds(i * 16, 16)
                idx_v[ck, 0, sl] = idx_v[ck, 0, sl] + off
            return c2

        lax.fori_loop(0, nech, off_loop, None)

        def z0(k, c2):
            s0 = row0 + k * _RCH
            pltpu.sync_copy(u0_hbm.at[pl.ds(off + s0, _RCH)], sbuf)
            pltpu.sync_copy(sbuf, u_hbm.at[pl.ds(off + s0, _RCH)])
            pltpu.sync_copy(zbuf, s_sh.at[pl.ds(s0, _RCH)])
            return c2

        lax.fori_loop(0, nrch, z0, None)
        plsc.subcore_barrier()

        def iter_body(it, carry):
            def outer(ot, c2):
                tb = ot * _NBUF
                for b in range(_NBUF):
                    @pl.when(ot > 0)
                    def _drain():
                        _scatter_cp(tb + b, b).wait()
                    _gather_cp(tb + b, b).start()
                for b in range(_NBUF):
                    _gather_cp(tb + b, b).wait()
                    _scatter_cp(tb + b, b).start(add=True)
                return c2

            lax.fori_loop(0, nech // _NBUF, outer, None)
            for b in range(_NBUF):
                _scatter_cp(b, b).wait()
            plsc.subcore_barrier()

            def echunk(k, c2):
                r0 = row0 + k * _RCH
                pltpu.sync_copy(s_sh.at[pl.ds(r0, _RCH)], sbuf)
                pltpu.sync_copy(zbuf, s_sh.at[pl.ds(r0, _RCH)])
                pltpu.sync_copy(resdis_hbm.at[cid, tid * nrch + k], rdbuf)

                def erow(j, c3):
                    for i in range(ch // 16):
                        sl = pl.ds(i * 16, 16)
                        dv = rdbuf[_RCH + j, sl]
                        sv = sbuf[j, sl]
                        rv = rdbuf[j, sl]
                        ov = jnp.minimum(
                            jnp.maximum(alpha * dv * sv + rv, lo), hi)
                        sbuf[j, sl] = ov
                        rdbuf[_RCH + j, sl] = dv * ov
                    return c3

                lax.fori_loop(0, _RCH, erow, None)
                pltpu.sync_copy(rdbuf.at[pl.ds(_RCH, _RCH)],
                                u_hbm.at[pl.ds(off + r0, _RCH)])

                @pl.when(it == iters - 1)
                def _write_out():
                    pltpu.sync_copy(sbuf, out_hbm.at[cid, pl.ds(r0, _RCH)])

                return c2

            lax.fori_loop(0, nrch, echunk, None)
            plsc.subcore_barrier()
            return carry

        lax.fori_loop(0, iters, iter_body, None)

    return pl.kernel(
        body,
        out_type=[jax.ShapeDtypeStruct((_NSC, n_pad, ch), _f32),
                  jax.ShapeDtypeStruct((_NSC * n_pad, ch), _f32)],
        mesh=mesh,
        compiler_params=pltpu.CompilerParams(use_tc_tiling_on_sc=False),
        scratch_types=[
            pltpu.VMEM_SHARED((n_pad, ch), _f32),
            pltpu.VMEM((nech, 2, _ECH), jnp.int32),
            pltpu.VMEM((_NBUF, _ECH, ch), _f32),
            pltpu.VMEM((_RCH, ch), _f32),
            pltpu.VMEM((_RCH, ch), _f32),
            pltpu.VMEM((2 * _RCH, ch), _f32),
            pltpu.SemaphoreType.DMA((_NBUF,)),
            pltpu.SemaphoreType.DMA((_NBUF,)),
        ],
    )


def _mm_body(x_ref, w_ref, o_ref):
    o_ref[...] = jnp.dot(x_ref[...], w_ref[...],
                         preferred_element_type=_f32)


def _pre_body(deg_ref, xw_ref, disb_ref, u0_ref):
    deg = deg_ref[...]
    dis = jnp.where(deg > 0, lax.rsqrt(jnp.maximum(deg, 1e-12)), 0.0)
    disb = jnp.broadcast_to(dis, xw_ref.shape)
    disb_ref[...] = disb
    u0_ref[...] = disb * xw_ref[...]


def _mid_body(logits_ref, maskf_ref, labels_ref, disb_ref,
              probs_ref, res_ref, u0_ref):
    n_pad, c = logits_ref.shape
    logits = logits_ref[...]
    m = jnp.max(logits, axis=-1, keepdims=True)
    e = jnp.exp(logits - m)
    probs = e / jnp.sum(e, axis=-1, keepdims=True)
    oh = (labels_ref[...] ==
          lax.broadcasted_iota(jnp.int32, (n_pad, c), 1)).astype(_f32)
    err = maskf_ref[...] * (oh - probs)
    probs_ref[...] = probs
    res_ref[...] = (1.0 - _CORRECTION_ALPHA) * err
    u0_ref[...] = disb_ref[...] * err


def _mid2_body(probs_ref, smerr_ref, maskf_ref, labels_ref, disb_ref,
               res_ref, u0_ref, n_nodes):
    n_pad, c = probs_ref.shape
    corrected = probs_ref[...] + _SCALE * smerr_ref[...]
    oh = (labels_ref[...] ==
          lax.broadcasted_iota(jnp.int32, (n_pad, c), 1)).astype(_f32)
    m = maskf_ref[...]
    valid = (lax.broadcasted_iota(jnp.int32, (n_pad, c), 0)
             < n_nodes).astype(_f32)
    y = (m * oh + (1.0 - m) * corrected) * valid
    res_ref[...] = (1.0 - _SMOOTHING_ALPHA) * y
    u0_ref[...] = disb_ref[...] * y


def _log_body(s_ref, o_ref):
    o_ref[...] = jnp.log(jnp.clip(s_ref[...], 1e-15, None))


def kernel(x, edge_index, W, train_mask, train_labels):
    n = x.shape[0]
    c = W.shape[1]
    e = edge_index.shape[1]
    src = edge_index[0]
    dst = edge_index[1]

    n_pad = -(-n // (_NTILES * _RCH)) * (_NTILES * _RCH)
    nech = -(-e // (_NTILES * _ECH))
    nech = -(-nech // _NBUF) * _NBUF
    e_pad = _NTILES * nech * _ECH
    rt = n_pad // _NTILES

    pad_cnt = e_pad - e
    pad_idx = n + (jnp.arange(pad_cnt, dtype=jnp.int32) % 8)
    src_t = jnp.concatenate([src, pad_idx]).reshape(_NTILES, nech, _ECH)
    dst_t = jnp.concatenate([dst, pad_idx]).reshape(_NTILES, nech, _ECH)
    srcdst = jnp.stack([src_t, dst_t], axis=2)

    x_p = jnp.pad(x, ((0, n_pad - n), (0, 0)))
    maskf = jnp.pad(train_mask.astype(_f32), (0, n_pad - n)).reshape(n_pad, 1)
    labels_p = jnp.pad(train_labels, (0, n_pad - n),
                       constant_values=-1).reshape(n_pad, 1)
    zeros_rc = jnp.zeros((_RCH, c), _f32)
    zeros_nc = jnp.zeros((n_pad, c), _f32)
    zeros_rt = jnp.zeros((rt,), _f32)
    ones_e = jnp.ones((_ECH,), _f32)

    deg = _make_deg_kernel(n_pad, nech)(srcdst, ones_e, zeros_rt)

    xw = pl.pallas_call(
        _mm_body, out_shape=jax.ShapeDtypeStruct((n_pad, c), _f32))(x_p, W)

    disb, u0_l = pl.pallas_call(
        _pre_body,
        out_shape=[jax.ShapeDtypeStruct((n_pad, c), _f32)] * 2,
    )(deg.reshape(n_pad, 1), xw)

    big = 3.0e38
    chalf = c // _NSC
    nb = n_pad // _RCH

    def split(a):
        return jnp.stack([a[:, :chalf], a[:, chalf:]])

    def join(a):
        return jnp.concatenate([a[0], a[1]], axis=1)

    def blockify(a):
        return a.reshape(_NSC, nb, _RCH, chalf)

    disb_b = blockify(split(disb))

    def resdis(res):
        return jnp.concatenate([blockify(split(res)), disb_b], axis=2)

    zeros_rd = resdis(jnp.zeros((n_pad, c), _f32))
    zeros_rc2 = jnp.zeros((_RCH, chalf), _f32)

    def stacku(a):
        return split(a).reshape(_NSC * n_pad, chalf)

    logits = join(_make_prop_kernel(n_pad, c, nech, 1, 1.0, -big, big)(
        srcdst, stacku(u0_l), zeros_rd, zeros_rc2)[0])

    probs, res_c, u0_c = pl.pallas_call(
        _mid_body,
        out_shape=[jax.ShapeDtypeStruct((n_pad, c), _f32)] * 3,
    )(logits, maskf, labels_p, disb)

    smerr = join(_make_prop_kernel(
        n_pad, c, nech, _CORRECTION_LAYERS, _CORRECTION_ALPHA, -1.0, 1.0)(
        srcdst, stacku(u0_c), resdis(res_c), zeros_rc2)[0])

    res_s, u0_s = pl.pallas_call(
        functools.partial(_mid2_body, n_nodes=n),
        out_shape=[jax.ShapeDtypeStruct((n_pad, c), _f32)] * 2,
    )(probs, smerr, maskf, labels_p, disb)

    smoothed = join(_make_prop_kernel(
        n_pad, c, nech, _SMOOTHING_LAYERS, _SMOOTHING_ALPHA, 0.0, 1.0)(
        srcdst, stacku(u0_s), resdis(res_s), zeros_rc2)[0])

    out = pl.pallas_call(
        _log_body, out_shape=jax.ShapeDtypeStruct((n_pad, c), _f32))(smoothed)
    return out[:n]


# speedup vs baseline: 1.2895x; 1.2895x over previous
import functools

import jax
import jax.numpy as jnp
from jax import lax
from jax.experimental import pallas as pl
from jax.experimental.pallas import tpu as pltpu
from jax.experimental.pallas import tpu_sc as plsc

_f32 = jnp.float32

_NTILES = 16
_RCH = 128
_ECH = 128

_CORRECTION_LAYERS = 50
_CORRECTION_ALPHA = 0.5
_SMOOTHING_LAYERS = 50
_SMOOTHING_ALPHA = 0.8
_SCALE = 1.0


def _make_deg_kernel(n_pad, nech):
    rt = n_pad // _NTILES
    mesh = plsc.VectorSubcoreMesh(
        core_axis_name="c", subcore_axis_name="s", num_cores=1, num_subcores=16)

    def body(srcdst_hbm, ones_hbm, zer_hbm, deg_hbm, deg_sh, dst_v, ones_v):
        tid = lax.axis_index("s")
        r0 = tid * rt
        pltpu.sync_copy(srcdst_hbm.at[tid], dst_v)
        pltpu.sync_copy(ones_hbm, ones_v)
        pltpu.sync_copy(zer_hbm, deg_sh.at[pl.ds(r0, rt)])
        plsc.subcore_barrier()

        def eloop(c, carry):
            pltpu.sync_copy(ones_v, deg_sh.at[dst_v.at[c, 1]], add=True)
            return carry

        lax.fori_loop(0, nech, eloop, None)
        plsc.subcore_barrier()
        pltpu.sync_copy(deg_sh.at[pl.ds(r0, rt)], deg_hbm.at[pl.ds(r0, rt)])

    return pl.kernel(
        body,
        out_type=jax.ShapeDtypeStruct((n_pad,), _f32),
        mesh=mesh,
        compiler_params=pltpu.CompilerParams(use_tc_tiling_on_sc=False),
        scratch_types=[
            pltpu.VMEM_SHARED((n_pad,), _f32),
            pltpu.VMEM((nech, 2, _ECH), jnp.int32),
            pltpu.VMEM((_ECH,), _f32),
        ],
    )


_NBUF = 7
_NSC = 2


def _make_prop_kernel(n_pad, c, nech, iters, alpha, lo, hi):
    ch = c // _NSC
    rt = n_pad // _NTILES
    nrch = rt // _RCH
    assert nech % _NBUF == 0
    mesh = plsc.VectorSubcoreMesh(
        core_axis_name="c", subcore_axis_name="s",
        num_cores=_NSC, num_subcores=16)

    def body(srcdst_hbm, u0_hbm, resdis_hbm, zer_hbm, out_hbm,
             u_sh, s_sh, idx_v, gbuf, zbuf, sbuf, rdbuf,
             gsem, ssem):
        cid = lax.axis_index("c")
        tid = lax.axis_index("s")
        row0 = tid * rt

        def _gather_cp(ck, b):
            return pltpu.make_async_copy(
                u_sh.at[idx_v.at[ck, 0]], gbuf.at[b], gsem.at[b])

        def _scatter_cp(ck, b):
            return pltpu.make_async_copy(
                gbuf.at[b], s_sh.at[idx_v.at[ck, 1]], ssem.at[b])

        pltpu.sync_copy(srcdst_hbm.at[tid], idx_v)
        pltpu.sync_copy(zer_hbm, zbuf)
        pltpu.sync_copy(u0_hbm.at[cid, pl.ds(row0, rt)],
                        u_sh.at[pl.ds(row0, rt)])

        def z0(k, c2):
            pltpu.sync_copy(zbuf, s_sh.at[pl.ds(row0 + k * _RCH, _RCH)])
            return c2

        lax.fori_loop(0, nrch, z0, None)
        plsc.subcore_barrier()

        def iter_body(it, carry):
            def outer(ot, c2):
                tb = ot * _NBUF
                for b in range(_NBUF):
                    @pl.when(ot > 0)
                    def _drain():
                        _scatter_cp(tb + b, b).wait()
                    _gather_cp(tb + b, b).start()
                for b in range(_NBUF):
                    _gather_cp(tb + b, b).wait()
                    _scatter_cp(tb + b, b).start(add=True)
                return c2

            lax.fori_loop(0, nech // _NBUF, outer, None)
            for b in range(_NBUF):
                _scatter_cp(b, b).wait()
            plsc.subcore_barrier()

            def echunk(k, c2):
                r0 = row0 + k * _RCH
                pltpu.sync_copy(s_sh.at[pl.ds(r0, _RCH)], sbuf)
                pltpu.sync_copy(zbuf, s_sh.at[pl.ds(r0, _RCH)])
                pltpu.sync_copy(resdis_hbm.at[cid, tid * nrch + k], rdbuf)

                def erow(j, c3):
                    for i in range(ch // 16):
                        sl = pl.ds(i * 16, 16)
                        dv = rdbuf[_RCH + j, sl]
                        sv = sbuf[j, sl]
                        rv = rdbuf[j, sl]
                        ov = jnp.minimum(
                            jnp.maximum(alpha * dv * sv + rv, lo), hi)
                        sbuf[j, sl] = ov
                        rdbuf[_RCH + j, sl] = dv * ov
                    return c3

                lax.fori_loop(0, _RCH, erow, None)
                pltpu.sync_copy(rdbuf.at[pl.ds(_RCH, _RCH)],
                                u_sh.at[pl.ds(r0, _RCH)])

                @pl.when(it == iters - 1)
                def _write_out():
                    pltpu.sync_copy(sbuf, out_hbm.at[cid, pl.ds(r0, _RCH)])

                return c2

            lax.fori_loop(0, nrch, echunk, None)
            plsc.subcore_barrier()
            return carry

        lax.fori_loop(0, iters, iter_body, None)

    return pl.kernel(
        body,
        out_type=jax.ShapeDtypeStruct((_NSC, n_pad, ch), _f32),
        mesh=mesh,
        compiler_params=pltpu.CompilerParams(use_tc_tiling_on_sc=False),
        scratch_types=[
            pltpu.VMEM_SHARED((n_pad, ch), _f32),
            pltpu.VMEM_SHARED((n_pad, ch), _f32),
            pltpu.VMEM((nech, 2, _ECH), jnp.int32),
            pltpu.VMEM((_NBUF, _ECH, ch), _f32),
            pltpu.VMEM((_RCH, ch), _f32),
            pltpu.VMEM((_RCH, ch), _f32),
            pltpu.VMEM((2 * _RCH, ch), _f32),
            pltpu.SemaphoreType.DMA((_NBUF,)),
            pltpu.SemaphoreType.DMA((_NBUF,)),
        ],
    )


def _mm_body(x_ref, w_ref, o_ref):
    o_ref[...] = jnp.dot(x_ref[...], w_ref[...],
                         preferred_element_type=_f32)


def _pre_body(deg_ref, xw_ref, disb_ref, u0_ref):
    deg = deg_ref[...]
    dis = jnp.where(deg > 0, lax.rsqrt(jnp.maximum(deg, 1e-12)), 0.0)
    disb = jnp.broadcast_to(dis, xw_ref.shape)
    disb_ref[...] = disb
    u0_ref[...] = disb * xw_ref[...]


def _mid_body(logits_ref, maskf_ref, labels_ref, disb_ref,
              probs_ref, res_ref, u0_ref):
    n_pad, c = logits_ref.shape
    logits = logits_ref[...]
    m = jnp.max(logits, axis=-1, keepdims=True)
    e = jnp.exp(logits - m)
    probs = e / jnp.sum(e, axis=-1, keepdims=True)
    oh = (labels_ref[...] ==
          lax.broadcasted_iota(jnp.int32, (n_pad, c), 1)).astype(_f32)
    err = maskf_ref[...] * (oh - probs)
    probs_ref[...] = probs
    res_ref[...] = (1.0 - _CORRECTION_ALPHA) * err
    u0_ref[...] = disb_ref[...] * err


def _mid2_body(probs_ref, smerr_ref, maskf_ref, labels_ref, disb_ref,
               res_ref, u0_ref, n_nodes):
    n_pad, c = probs_ref.shape
    corrected = probs_ref[...] + _SCALE * smerr_ref[...]
    oh = (labels_ref[...] ==
          lax.broadcasted_iota(jnp.int32, (n_pad, c), 1)).astype(_f32)
    m = maskf_ref[...]
    valid = (lax.broadcasted_iota(jnp.int32, (n_pad, c), 0)
             < n_nodes).astype(_f32)
    y = (m * oh + (1.0 - m) * corrected) * valid
    res_ref[...] = (1.0 - _SMOOTHING_ALPHA) * y
    u0_ref[...] = disb_ref[...] * y


def _log_body(s_ref, o_ref):
    o_ref[...] = jnp.log(jnp.clip(s_ref[...], 1e-15, None))


def kernel(x, edge_index, W, train_mask, train_labels):
    n = x.shape[0]
    c = W.shape[1]
    e = edge_index.shape[1]
    src = edge_index[0]
    dst = edge_index[1]

    n_pad = -(-n // (_NTILES * _RCH)) * (_NTILES * _RCH)
    nech = -(-e // (_NTILES * _ECH))
    nech = -(-nech // _NBUF) * _NBUF
    e_pad = _NTILES * nech * _ECH
    rt = n_pad // _NTILES

    pad_cnt = e_pad - e
    pad_idx = n + (jnp.arange(pad_cnt, dtype=jnp.int32) % 8)
    src_t = jnp.concatenate([src, pad_idx]).reshape(_NTILES, nech, _ECH)
    dst_t = jnp.concatenate([dst, pad_idx]).reshape(_NTILES, nech, _ECH)
    srcdst = jnp.stack([src_t, dst_t], axis=2)

    x_p = jnp.pad(x, ((0, n_pad - n), (0, 0)))
    maskf = jnp.pad(train_mask.astype(_f32), (0, n_pad - n)).reshape(n_pad, 1)
    labels_p = jnp.pad(train_labels, (0, n_pad - n),
                       constant_values=-1).reshape(n_pad, 1)
    zeros_rc = jnp.zeros((_RCH, c), _f32)
    zeros_nc = jnp.zeros((n_pad, c), _f32)
    zeros_rt = jnp.zeros((rt,), _f32)
    ones_e = jnp.ones((_ECH,), _f32)

    deg = _make_deg_kernel(n_pad, nech)(srcdst, ones_e, zeros_rt)

    xw = pl.pallas_call(
        _mm_body, out_shape=jax.ShapeDtypeStruct((n_pad, c), _f32))(x_p, W)

    disb, u0_l = pl.pallas_call(
        _pre_body,
        out_shape=[jax.ShapeDtypeStruct((n_pad, c), _f32)] * 2,
    )(deg.reshape(n_pad, 1), xw)

    big = 3.0e38
    chalf = c // _NSC
    nb = n_pad // _RCH

    def split(a):
        return jnp.stack([a[:, :chalf], a[:, chalf:]])

    def join(a):
        return jnp.concatenate([a[0], a[1]], axis=1)

    def blockify(a):
        return a.reshape(_NSC, nb, _RCH, chalf)

    disb_b = blockify(split(disb))

    def resdis(res):
        return jnp.concatenate([blockify(split(res)), disb_b], axis=2)

    zeros_rd = resdis(jnp.zeros((n_pad, c), _f32))
    zeros_rc2 = jnp.zeros((_RCH, chalf), _f32)

    logits = join(_make_prop_kernel(n_pad, c, nech, 1, 1.0, -big, big)(
        srcdst, split(u0_l), zeros_rd, zeros_rc2))

    probs, res_c, u0_c = pl.pallas_call(
        _mid_body,
        out_shape=[jax.ShapeDtypeStruct((n_pad, c), _f32)] * 3,
    )(logits, maskf, labels_p, disb)

    smerr = join(_make_prop_kernel(
        n_pad, c, nech, _CORRECTION_LAYERS, _CORRECTION_ALPHA, -1.0, 1.0)(
        srcdst, split(u0_c), resdis(res_c), zeros_rc2))

    res_s, u0_s = pl.pallas_call(
        functools.partial(_mid2_body, n_nodes=n),
        out_shape=[jax.ShapeDtypeStruct((n_pad, c), _f32)] * 2,
    )(probs, smerr, maskf, labels_p, disb)

    smoothed = join(_make_prop_kernel(
        n_pad, c, nech, _SMOOTHING_LAYERS, _SMOOTHING_ALPHA, 0.0, 1.0)(
        srcdst, split(u0_s), resdis(res_s), zeros_rc2))

    out = pl.pallas_call(
        _log_body, out_shape=jax.ShapeDtypeStruct((n_pad, c), _f32))(smoothed)
    return out[:n]


# speedup vs baseline: 1.3833x; 1.0727x over previous
import functools

import jax
import jax.numpy as jnp
from jax import lax
from jax.experimental import pallas as pl
from jax.experimental.pallas import tpu as pltpu
from jax.experimental.pallas import tpu_sc as plsc

_f32 = jnp.float32

_NTILES = 16
_RCH = 128
_ECH = 128

_CORRECTION_LAYERS = 50
_CORRECTION_ALPHA = 0.5
_SMOOTHING_LAYERS = 50
_SMOOTHING_ALPHA = 0.8
_SCALE = 1.0


def _make_deg_kernel(n_pad, nech):
    rt = n_pad // _NTILES
    mesh = plsc.VectorSubcoreMesh(
        core_axis_name="c", subcore_axis_name="s", num_cores=1, num_subcores=16)

    def body(srcdst_hbm, ones_hbm, zer_hbm, deg_hbm, deg_sh, dst_v, ones_v):
        tid = lax.axis_index("s")
        r0 = tid * rt
        pltpu.sync_copy(srcdst_hbm.at[tid], dst_v)
        pltpu.sync_copy(ones_hbm, ones_v)
        pltpu.sync_copy(zer_hbm, deg_sh.at[pl.ds(r0, rt)])
        plsc.subcore_barrier()

        def eloop(c, carry):
            pltpu.sync_copy(ones_v, deg_sh.at[dst_v.at[c, 1]], add=True)
            return carry

        lax.fori_loop(0, nech, eloop, None)
        plsc.subcore_barrier()
        pltpu.sync_copy(deg_sh.at[pl.ds(r0, rt)], deg_hbm.at[pl.ds(r0, rt)])

    return pl.kernel(
        body,
        out_type=jax.ShapeDtypeStruct((n_pad,), _f32),
        mesh=mesh,
        compiler_params=pltpu.CompilerParams(use_tc_tiling_on_sc=False),
        scratch_types=[
            pltpu.VMEM_SHARED((n_pad,), _f32),
            pltpu.VMEM((nech, 2, _ECH), jnp.int32),
            pltpu.VMEM((_ECH,), _f32),
        ],
    )


_NBUF = 6
_NSC = 2


def _make_prop_kernel(n_pad, c, nech, iters, alpha, lo, hi):
    ch = c // _NSC
    rt = n_pad // _NTILES
    nrch = rt // _RCH
    assert nech % _NBUF == 0
    mesh = plsc.VectorSubcoreMesh(
        core_axis_name="c", subcore_axis_name="s",
        num_cores=_NSC, num_subcores=16)

    def body(srcdst_hbm, u0_hbm, resdis_hbm, zer_hbm, out_hbm,
             u_sh, s_sh, idx_v, gbuf, zbuf, sbuf, rdbuf,
             gsem, ssem):
        cid = lax.axis_index("c")
        tid = lax.axis_index("s")
        row0 = tid * rt

        def _gather_cp(ck, b):
            return pltpu.make_async_copy(
                u_sh.at[idx_v.at[ck, 0]], gbuf.at[b], gsem.at[b])

        def _scatter_cp(ck, b):
            return pltpu.make_async_copy(
                gbuf.at[b], s_sh.at[idx_v.at[ck, 1]], ssem.at[b])

        pltpu.sync_copy(srcdst_hbm.at[tid], idx_v)
        pltpu.sync_copy(zer_hbm, zbuf)
        pltpu.sync_copy(u0_hbm.at[cid, pl.ds(row0, rt)],
                        u_sh.at[pl.ds(row0, rt)])

        def z0(k, c2):
            pltpu.sync_copy(zbuf, s_sh.at[pl.ds(row0 + k * _RCH, _RCH)])
            return c2

        lax.fori_loop(0, nrch, z0, None)
        plsc.subcore_barrier()

        def iter_body(it, carry):
            def outer(ot, c2):
                tb = ot * _NBUF
                for b in range(_NBUF):
                    @pl.when(ot > 0)
                    def _drain():
                        _scatter_cp(tb + b, b).wait()
                    _gather_cp(tb + b, b).start()
                for b in range(_NBUF):
                    _gather_cp(tb + b, b).wait()
                    _scatter_cp(tb + b, b).start(add=True)
                return c2

            lax.fori_loop(0, nech // _NBUF, outer, None)
            for b in range(_NBUF):
                _scatter_cp(b, b).wait()
            plsc.subcore_barrier()

            def echunk(k, c2):
                r0 = row0 + k * _RCH
                pltpu.sync_copy(s_sh.at[pl.ds(r0, _RCH)], sbuf)
                pltpu.sync_copy(zbuf, s_sh.at[pl.ds(r0, _RCH)])
                pltpu.sync_copy(resdis_hbm.at[cid, tid * nrch + k], rdbuf)

                def erow(j4, c3):
                    for jj in range(4):
                        j = j4 * 4 + jj
                        for i in range(ch // 16):
                            sl = pl.ds(i * 16, 16)
                            dv = rdbuf[_RCH + j, sl]
                            sv = sbuf[j, sl]
                            rv = rdbuf[j, sl]
                            ov = jnp.minimum(
                                jnp.maximum(alpha * dv * sv + rv, lo), hi)
                            sbuf[j, sl] = ov
                            rdbuf[_RCH + j, sl] = dv * ov
                    return c3

                lax.fori_loop(0, _RCH // 4, erow, None)
                pltpu.sync_copy(rdbuf.at[pl.ds(_RCH, _RCH)],
                                u_sh.at[pl.ds(r0, _RCH)])

                @pl.when(it == iters - 1)
                def _write_out():
                    pltpu.sync_copy(sbuf, out_hbm.at[cid, pl.ds(r0, _RCH)])

                return c2

            lax.fori_loop(0, nrch, echunk, None)
            plsc.subcore_barrier()
            return carry

        lax.fori_loop(0, iters, iter_body, None)

    return pl.kernel(
        body,
        out_type=jax.ShapeDtypeStruct((_NSC, n_pad, ch), _f32),
        mesh=mesh,
        compiler_params=pltpu.CompilerParams(use_tc_tiling_on_sc=False),
        scratch_types=[
            pltpu.VMEM_SHARED((n_pad, ch), _f32),
            pltpu.VMEM_SHARED((n_pad, ch), _f32),
            pltpu.VMEM((nech, 2, _ECH), jnp.int32),
            pltpu.VMEM((_NBUF, _ECH, ch), _f32),
            pltpu.VMEM((_RCH, ch), _f32),
            pltpu.VMEM((_RCH, ch), _f32),
            pltpu.VMEM((2 * _RCH, ch), _f32),
            pltpu.SemaphoreType.DMA((_NBUF,)),
            pltpu.SemaphoreType.DMA((_NBUF,)),
        ],
    )


def _mm_body(x_ref, w_ref, o_ref):
    o_ref[...] = jnp.dot(x_ref[...], w_ref[...],
                         preferred_element_type=_f32)


def _pre_body(deg_ref, xw_ref, disb_ref, u0_ref):
    deg = deg_ref[...]
    dis = jnp.where(deg > 0, lax.rsqrt(jnp.maximum(deg, 1e-12)), 0.0)
    disb = jnp.broadcast_to(dis, xw_ref.shape)
    disb_ref[...] = disb
    u0_ref[...] = disb * xw_ref[...]


def _mid_body(logits_ref, maskf_ref, labels_ref, disb_ref,
              probs_ref, res_ref, u0_ref):
    n_pad, c = logits_ref.shape
    logits = logits_ref[...]
    m = jnp.max(logits, axis=-1, keepdims=True)
    e = jnp.exp(logits - m)
    probs = e / jnp.sum(e, axis=-1, keepdims=True)
    oh = (labels_ref[...] ==
          lax.broadcasted_iota(jnp.int32, (n_pad, c), 1)).astype(_f32)
    err = maskf_ref[...] * (oh - probs)
    probs_ref[...] = probs
    res_ref[...] = (1.0 - _CORRECTION_ALPHA) * err
    u0_ref[...] = disb_ref[...] * err


def _mid2_body(probs_ref, smerr_ref, maskf_ref, labels_ref, disb_ref,
               res_ref, u0_ref, n_nodes):
    n_pad, c = probs_ref.shape
    corrected = probs_ref[...] + _SCALE * smerr_ref[...]
    oh = (labels_ref[...] ==
          lax.broadcasted_iota(jnp.int32, (n_pad, c), 1)).astype(_f32)
    m = maskf_ref[...]
    valid = (lax.broadcasted_iota(jnp.int32, (n_pad, c), 0)
             < n_nodes).astype(_f32)
    y = (m * oh + (1.0 - m) * corrected) * valid
    res_ref[...] = (1.0 - _SMOOTHING_ALPHA) * y
    u0_ref[...] = disb_ref[...] * y


def _log_body(s_ref, o_ref):
    o_ref[...] = jnp.log(jnp.clip(s_ref[...], 1e-15, None))


def kernel(x, edge_index, W, train_mask, train_labels):
    n = x.shape[0]
    c = W.shape[1]
    e = edge_index.shape[1]
    src = edge_index[0]
    dst = edge_index[1]

    n_pad = -(-n // (_NTILES * _RCH)) * (_NTILES * _RCH)
    nech = -(-e // (_NTILES * _ECH))
    nech = -(-nech // _NBUF) * _NBUF
    e_pad = _NTILES * nech * _ECH
    rt = n_pad // _NTILES

    pad_cnt = e_pad - e
    pad_idx = n + (jnp.arange(pad_cnt, dtype=jnp.int32) % 8)
    src_t = jnp.concatenate([src, pad_idx]).reshape(_NTILES, nech, _ECH)
    dst_t = jnp.concatenate([dst, pad_idx]).reshape(_NTILES, nech, _ECH)
    srcdst = jnp.stack([src_t, dst_t], axis=2)

    x_p = jnp.pad(x, ((0, n_pad - n), (0, 0)))
    maskf = jnp.pad(train_mask.astype(_f32), (0, n_pad - n)).reshape(n_pad, 1)
    labels_p = jnp.pad(train_labels, (0, n_pad - n),
                       constant_values=-1).reshape(n_pad, 1)
    zeros_rc = jnp.zeros((_RCH, c), _f32)
    zeros_nc = jnp.zeros((n_pad, c), _f32)
    zeros_rt = jnp.zeros((rt,), _f32)
    ones_e = jnp.ones((_ECH,), _f32)

    deg = _make_deg_kernel(n_pad, nech)(srcdst, ones_e, zeros_rt)

    xw = pl.pallas_call(
        _mm_body, out_shape=jax.ShapeDtypeStruct((n_pad, c), _f32))(x_p, W)

    disb, u0_l = pl.pallas_call(
        _pre_body,
        out_shape=[jax.ShapeDtypeStruct((n_pad, c), _f32)] * 2,
    )(deg.reshape(n_pad, 1), xw)

    big = 3.0e38
    chalf = c // _NSC
    nb = n_pad // _RCH

    def split(a):
        return jnp.stack([a[:, :chalf], a[:, chalf:]])

    def join(a):
        return jnp.concatenate([a[0], a[1]], axis=1)

    def blockify(a):
        return a.reshape(_NSC, nb, _RCH, chalf)

    disb_b = blockify(split(disb))

    def resdis(res):
        return jnp.concatenate([blockify(split(res)), disb_b], axis=2)

    zeros_rd = resdis(jnp.zeros((n_pad, c), _f32))
    zeros_rc2 = jnp.zeros((_RCH, chalf), _f32)

    logits = join(_make_prop_kernel(n_pad, c, nech, 1, 1.0, -big, big)(
        srcdst, split(u0_l), zeros_rd, zeros_rc2))

    probs, res_c, u0_c = pl.pallas_call(
        _mid_body,
        out_shape=[jax.ShapeDtypeStruct((n_pad, c), _f32)] * 3,
    )(logits, maskf, labels_p, disb)

    smerr = join(_make_prop_kernel(
        n_pad, c, nech, _CORRECTION_LAYERS, _CORRECTION_ALPHA, -1.0, 1.0)(
        srcdst, split(u0_c), resdis(res_c), zeros_rc2))

    res_s, u0_s = pl.pallas_call(
        functools.partial(_mid2_body, n_nodes=n),
        out_shape=[jax.ShapeDtypeStruct((n_pad, c), _f32)] * 2,
    )(probs, smerr, maskf, labels_p, disb)

    smoothed = join(_make_prop_kernel(
        n_pad, c, nech, _SMOOTHING_LAYERS, _SMOOTHING_ALPHA, 0.0, 1.0)(
        srcdst, split(u0_s), resdis(res_s), zeros_rc2))

    out = pl.pallas_call(
        _log_body, out_shape=jax.ShapeDtypeStruct((n_pad, c), _f32))(smoothed)
    return out[:n]
